# baseline (speedup 1.0000x reference)
import functools

import jax
import jax.numpy as jnp
from jax.experimental import pallas as pl
from jax.experimental.pallas import tpu as pltpu

_BLK = 3072


def _copy_scatter_body(x_ref, o_ref, *, bump_rows):
    i = pl.program_id(0)
    o_ref[...] = x_ref[...]

    for row in bump_rows:
        @pl.when(i == row // _BLK)
        def _(row=row):
            r = row % _BLK
            o_ref[pl.ds(r, 1), :] = o_ref[pl.ds(r, 1), :] + jnp.float32(1.0)


def kernel(x):
    n, s, d = x.shape
    bump_rows = (0, n + 1)
    y = jnp.transpose(x, (1, 0, 2)).reshape(s * n, d)
    out = pl.pallas_call(
        functools.partial(_copy_scatter_body, bump_rows=bump_rows),
        out_shape=jax.ShapeDtypeStruct((s * n, d), x.dtype),
        grid=(s * n // _BLK,),
        in_specs=[pl.BlockSpec((_BLK, d), lambda i: (i, 0))],
        out_specs=pl.BlockSpec((_BLK, d), lambda i: (i, 0)),
        compiler_params=pltpu.CompilerParams(dimension_semantics=("parallel",)),
    )(y)
    return jnp.transpose(out.reshape(s, n, d), (1, 0, 2))
